# baseline (device time: 254612 ns/iter reference)
import jax
import jax.numpy as jnp
from jax import lax
from jax.experimental import pallas as pl
from jax.experimental.pallas import tpu as pltpu

N_DEV = 32


def kernel(x, w_mat, scale_x, scale_w):
    m_per, k = x.shape
    n = w_mat.shape[1]

    x8 = x.astype(jnp.float8_e4m3fn)
    w16 = w_mat.astype(jnp.bfloat16)
    s = (scale_x * scale_w).astype(jnp.float32)

    def body(x_ref, w_ref, s_ref, out_ref, comm_ref, send_sems, recv_sems):
        my = lax.axis_index("i")
        left = lax.rem(my + N_DEV - 1, N_DEV)
        right = lax.rem(my + 1, N_DEV)

        barrier = pltpu.get_barrier_semaphore()
        for nbr in (left, right):
            pl.semaphore_signal(
                barrier, inc=1,
                device_id=(nbr,), device_id_type=pl.DeviceIdType.MESH,
            )
        pl.semaphore_wait(barrier, 2)

        w = w_ref[...]
        scale = s_ref[0]

        def store_chunk(origin, chunk8):
            acc = jnp.dot(
                chunk8.astype(jnp.bfloat16), w,
                preferred_element_type=jnp.float32,
            )
            y = jnp.maximum(acc * scale, 0.0)
            out_ref[pl.ds(origin * m_per, m_per), :] = y

        store_chunk(my, x_ref[...])

        for h in range(N_DEV - 1):
            src = x_ref if h == 0 else comm_ref.at[h - 1]
            rdma = pltpu.make_async_remote_copy(
                src_ref=src,
                dst_ref=comm_ref.at[h],
                send_sem=send_sems.at[h],
                recv_sem=recv_sems.at[h],
                device_id=(right,),
                device_id_type=pl.DeviceIdType.MESH,
            )
            rdma.start()
            rdma.wait()

            origin = lax.rem(my + 2 * N_DEV - h - 1, N_DEV)
            store_chunk(origin, comm_ref[h])

    return pl.pallas_call(
        body,
        out_shape=jax.ShapeDtypeStruct((N_DEV * m_per, n), jnp.float32),
        in_specs=[
            pl.BlockSpec(memory_space=pltpu.VMEM),
            pl.BlockSpec(memory_space=pltpu.VMEM),
            pl.BlockSpec(memory_space=pltpu.SMEM),
        ],
        out_specs=pl.BlockSpec(memory_space=pltpu.VMEM),
        scratch_shapes=[
            pltpu.VMEM((N_DEV - 1, m_per, k), jnp.float8_e4m3fn),
            pltpu.SemaphoreType.DMA((N_DEV - 1,)),
            pltpu.SemaphoreType.DMA((N_DEV - 1,)),
        ],
        compiler_params=pltpu.CompilerParams(collective_id=0),
    )(x8, w16, s)


# device time: 132360 ns/iter; 1.9236x vs baseline; 1.9236x over previous
import jax
import jax.numpy as jnp
from jax import lax
from jax.experimental import pallas as pl
from jax.experimental.pallas import tpu as pltpu

N_DEV = 32
HR = 16
HL = 15


def _ring_order():
    ring = []
    for p in range(N_DEV):
        if p < 16:
            x, pp = 0, p
        else:
            x, pp = 1, 31 - p
        z = pp // 4
        yy = pp % 4
        y = yy if z % 2 == 0 else 3 - yy
        ring.append(z * 8 + y * 2 + (x if y % 2 == 0 else 1 - x))
    return ring


_RING = _ring_order()
_INV = [0] * N_DEV
for _p, _l in enumerate(_RING):
    _INV[_l] = _p


def kernel(x, w_mat, scale_x, scale_w):
    m_per, k = x.shape
    n = w_mat.shape[1]

    x8 = x.astype(jnp.float8_e4m3fn)
    w16 = w_mat.astype(jnp.bfloat16)
    s = (scale_x * scale_w).astype(jnp.float32)

    ring_arr = jnp.array(_RING, jnp.int32)
    inv_arr = jnp.array(_INV, jnp.int32)
    my = lax.axis_index("i")
    p = inv_arr[my]
    right = ring_arr[(p + 1) % N_DEV]
    left = ring_arr[(p - 1) % N_DEV]
    origins_r = ring_arr[(p - 1 - jnp.arange(HR)) % N_DEV]
    origins_l = ring_arr[(p + 1 + jnp.arange(HL)) % N_DEV]
    params = jnp.concatenate(
        [right[None], left[None], origins_r, origins_l]
    ).astype(jnp.int32)

    def body(x_ref, w_ref, s_ref, prm, out_ref,
             comm_r, comm_l, send_r, recv_r, send_l, recv_l):
        rt = prm[0]
        lt = prm[1]

        barrier = pltpu.get_barrier_semaphore()
        for nbr in (lt, rt):
            pl.semaphore_signal(
                barrier, inc=1,
                device_id=(nbr,), device_id_type=pl.DeviceIdType.MESH,
            )
        pl.semaphore_wait(barrier, 2)

        w = w_ref[...]
        scale = s_ref[0]

        def store_chunk(origin, chunk8):
            acc = jnp.dot(
                chunk8.astype(jnp.bfloat16), w,
                preferred_element_type=jnp.float32,
            )
            out_ref[pl.ds(origin * m_per, m_per), :] = jnp.maximum(
                acc * scale, 0.0
            )

        def mk(direction, h):
            if direction == "r":
                src = x_ref if h == 0 else comm_r.at[h - 1]
                return pltpu.make_async_remote_copy(
                    src_ref=src, dst_ref=comm_r.at[h],
                    send_sem=send_r.at[h], recv_sem=recv_r.at[h],
                    device_id=(rt,), device_id_type=pl.DeviceIdType.MESH,
                )
            src = x_ref if h == 0 else comm_l.at[h - 1]
            return pltpu.make_async_remote_copy(
                src_ref=src, dst_ref=comm_l.at[h],
                send_sem=send_l.at[h], recv_sem=recv_l.at[h],
                device_id=(lt,), device_id_type=pl.DeviceIdType.MESH,
            )

        rdma_r = [mk("r", 0)]
        rdma_l = [mk("l", 0)]
        rdma_r[0].start()
        rdma_l[0].start()

        store_chunk(lax.axis_index("i"), x_ref[...])

        for h in range(HR):
            rdma_r[h].wait_recv()
            if h + 1 < HR:
                rdma_r.append(mk("r", h + 1))
                rdma_r[h + 1].start()
            if h < HL:
                rdma_l[h].wait_recv()
                if h + 1 < HL:
                    rdma_l.append(mk("l", h + 1))
                    rdma_l[h + 1].start()
            store_chunk(prm[2 + h], comm_r[h])
            if h < HL:
                store_chunk(prm[2 + HR + h], comm_l[h])

        for d in rdma_r + rdma_l:
            d.wait_send()

    return pl.pallas_call(
        body,
        out_shape=jax.ShapeDtypeStruct((N_DEV * m_per, n), jnp.float32),
        in_specs=[
            pl.BlockSpec(memory_space=pltpu.VMEM),
            pl.BlockSpec(memory_space=pltpu.VMEM),
            pl.BlockSpec(memory_space=pltpu.SMEM),
            pl.BlockSpec(memory_space=pltpu.SMEM),
        ],
        out_specs=pl.BlockSpec(memory_space=pltpu.VMEM),
        scratch_shapes=[
            pltpu.VMEM((HR, m_per, k), jnp.float8_e4m3fn),
            pltpu.VMEM((HL, m_per, k), jnp.float8_e4m3fn),
            pltpu.SemaphoreType.DMA((HR,)),
            pltpu.SemaphoreType.DMA((HR,)),
            pltpu.SemaphoreType.DMA((HL,)),
            pltpu.SemaphoreType.DMA((HL,)),
        ],
        compiler_params=pltpu.CompilerParams(collective_id=0),
    )(x8, w16, s, params)


# device time: 106152 ns/iter; 2.3986x vs baseline; 1.2469x over previous
import jax
import jax.numpy as jnp
from jax import lax
from jax.experimental import pallas as pl
from jax.experimental.pallas import tpu as pltpu

N_DEV = 32
HR = 16
HL = 15


def _ring_order():
    ring = []
    for p in range(N_DEV):
        if p < 16:
            x, pp = 0, p
        else:
            x, pp = 1, 31 - p
        z = pp // 4
        yy = pp % 4
        y = yy if z % 2 == 0 else 3 - yy
        ring.append(z * 8 + y * 2 + (x if y % 2 == 0 else 1 - x))
    return ring


_RING = _ring_order()
_INV = [0] * N_DEV
for _p, _l in enumerate(_RING):
    _INV[_l] = _p


def kernel(x, w_mat, scale_x, scale_w):
    m_per, k = x.shape
    n = w_mat.shape[1]

    x8 = x.astype(jnp.float8_e4m3fn)
    w16 = w_mat.astype(jnp.bfloat16)
    s = (scale_x * scale_w).astype(jnp.float32)

    ring_arr = jnp.array(_RING, jnp.int32)
    inv_arr = jnp.array(_INV, jnp.int32)
    my = lax.axis_index("i")
    p = inv_arr[my]
    right = ring_arr[(p + 1) % N_DEV]
    left = ring_arr[(p - 1) % N_DEV]
    origins_r = ring_arr[(p - 1 - jnp.arange(HR)) % N_DEV]
    origins_l = ring_arr[(p + 1 + jnp.arange(HL)) % N_DEV]
    params = jnp.concatenate(
        [right[None], left[None], origins_r, origins_l]
    ).astype(jnp.int32)

    def body(x_ref, w_ref, s_ref, prm, out_ref,
             comm_r, comm_l, send_r, recv_r, send_l, recv_l):
        rt = prm[0]
        lt = prm[1]

        barrier = pltpu.get_barrier_semaphore()
        for nbr in (lt, rt):
            pl.semaphore_signal(
                barrier, inc=1,
                device_id=(nbr,), device_id_type=pl.DeviceIdType.MESH,
            )
        pl.semaphore_wait(barrier, 2)

        w = w_ref[...]
        scale = s_ref[0]

        def store_chunk(origin, chunk8):
            acc = jnp.dot(
                chunk8.astype(jnp.bfloat16), w,
                preferred_element_type=jnp.float32,
            )
            out_ref[pl.ds(origin * m_per, m_per), :] = jnp.maximum(
                acc * scale, 0.0
            )

        half = m_per // 2
        subs = (pl.ds(0, half), pl.ds(half, half))

        def mk(direction, h, s):
            comm, send, recv, tgt = (
                (comm_r, send_r, recv_r, rt) if direction == "r"
                else (comm_l, send_l, recv_l, lt)
            )
            src = x_ref.at[subs[s]] if h == 0 else comm.at[h - 1, subs[s]]
            return pltpu.make_async_remote_copy(
                src_ref=src, dst_ref=comm.at[h, subs[s]],
                send_sem=send.at[h, s], recv_sem=recv.at[h, s],
                device_id=(tgt,), device_id_type=pl.DeviceIdType.MESH,
            )

        rdma_r = {(0, s): mk("r", 0, s) for s in (0, 1)}
        rdma_l = {(0, s): mk("l", 0, s) for s in (0, 1)}
        for s in (0, 1):
            rdma_r[0, s].start()
            rdma_l[0, s].start()

        store_chunk(lax.axis_index("i"), x_ref[...])

        for h in range(HR):
            for s in (0, 1):
                rdma_r[h, s].wait_recv()
                if h + 1 < HR:
                    rdma_r[h + 1, s] = mk("r", h + 1, s)
                    rdma_r[h + 1, s].start()
                if h < HL:
                    rdma_l[h, s].wait_recv()
                    if h + 1 < HL:
                        rdma_l[h + 1, s] = mk("l", h + 1, s)
                        rdma_l[h + 1, s].start()
            store_chunk(prm[2 + h], comm_r[h])
            if h < HL:
                store_chunk(prm[2 + HR + h], comm_l[h])

        for d in list(rdma_r.values()) + list(rdma_l.values()):
            d.wait_send()

    return pl.pallas_call(
        body,
        out_shape=jax.ShapeDtypeStruct((N_DEV * m_per, n), jnp.float32),
        in_specs=[
            pl.BlockSpec(memory_space=pltpu.VMEM),
            pl.BlockSpec(memory_space=pltpu.VMEM),
            pl.BlockSpec(memory_space=pltpu.SMEM),
            pl.BlockSpec(memory_space=pltpu.SMEM),
        ],
        out_specs=pl.BlockSpec(memory_space=pltpu.VMEM),
        scratch_shapes=[
            pltpu.VMEM((HR, m_per, k), jnp.float8_e4m3fn),
            pltpu.VMEM((HL, m_per, k), jnp.float8_e4m3fn),
            pltpu.SemaphoreType.DMA((HR, 2)),
            pltpu.SemaphoreType.DMA((HR, 2)),
            pltpu.SemaphoreType.DMA((HL, 2)),
            pltpu.SemaphoreType.DMA((HL, 2)),
        ],
        compiler_params=pltpu.CompilerParams(collective_id=0),
    )(x8, w16, s, params)


# device time: 105205 ns/iter; 2.4202x vs baseline; 1.0090x over previous
import jax
import jax.numpy as jnp
from jax import lax
from jax.experimental import pallas as pl
from jax.experimental.pallas import tpu as pltpu

N_DEV = 32
HR = 16
HL = 15


def _ring_order():
    ring = []
    for p in range(N_DEV):
        if p < 16:
            x, pp = 0, p
        else:
            x, pp = 1, 31 - p
        z = pp // 4
        yy = pp % 4
        y = yy if z % 2 == 0 else 3 - yy
        ring.append(z * 8 + y * 2 + (x if y % 2 == 0 else 1 - x))
    return ring


_RING = _ring_order()
_INV = [0] * N_DEV
for _p, _l in enumerate(_RING):
    _INV[_l] = _p


def kernel(x, w_mat, scale_x, scale_w):
    m_per, k = x.shape
    n = w_mat.shape[1]

    x8 = x.astype(jnp.float8_e4m3fn)
    w16 = w_mat.astype(jnp.bfloat16)
    s = (scale_x * scale_w).astype(jnp.float32)

    ring_arr = jnp.array(_RING, jnp.int32)
    inv_arr = jnp.array(_INV, jnp.int32)
    my = lax.axis_index("i")
    p = inv_arr[my]
    right = ring_arr[(p + 1) % N_DEV]
    left = ring_arr[(p - 1) % N_DEV]
    origins_r = ring_arr[(p - 1 - jnp.arange(HR)) % N_DEV]
    origins_l = ring_arr[(p + 1 + jnp.arange(HL)) % N_DEV]
    params = jnp.concatenate(
        [right[None], left[None], origins_r, origins_l]
    ).astype(jnp.int32)

    def body(x_ref, w_ref, s_ref, prm, out_ref,
             comm_r, comm_l, send_r, recv_r, send_l, recv_l):
        rt = prm[0]
        lt = prm[1]

        barrier = pltpu.get_barrier_semaphore()
        for nbr in (lt, rt):
            pl.semaphore_signal(
                barrier, inc=1,
                device_id=(nbr,), device_id_type=pl.DeviceIdType.MESH,
            )
        pl.semaphore_wait(barrier, 2)

        w = w_ref[...]
        scale = s_ref[0]

        def store_chunk(origin, chunk8):
            acc = jnp.dot(
                chunk8.astype(jnp.bfloat16), w,
                preferred_element_type=jnp.float32,
            )
            out_ref[pl.ds(origin * m_per, m_per), :] = jnp.maximum(
                acc * scale, 0.0
            )

        n_sub = 4
        half = m_per // n_sub
        subs = tuple(pl.ds(i * half, half) for i in range(n_sub))

        def mk(direction, h, s):
            comm, send, recv, tgt = (
                (comm_r, send_r, recv_r, rt) if direction == "r"
                else (comm_l, send_l, recv_l, lt)
            )
            src = x_ref.at[subs[s]] if h == 0 else comm.at[h - 1, subs[s]]
            return pltpu.make_async_remote_copy(
                src_ref=src, dst_ref=comm.at[h, subs[s]],
                send_sem=send.at[s], recv_sem=recv.at[s],
                device_id=(tgt,), device_id_type=pl.DeviceIdType.MESH,
            )

        rdma_r = {(0, s): mk("r", 0, s) for s in range(n_sub)}
        rdma_l = {(0, s): mk("l", 0, s) for s in range(n_sub)}
        for s in range(n_sub):
            rdma_r[0, s].start()
            rdma_l[0, s].start()

        store_chunk(lax.axis_index("i"), x_ref[...])

        for h in range(HR):
            for s in range(n_sub):
                rdma_r[h, s].wait_recv()
                if h + 1 < HR:
                    rdma_r[h + 1, s] = mk("r", h + 1, s)
                    rdma_r[h + 1, s].start()
                if h < HL:
                    rdma_l[h, s].wait_recv()
                    if h + 1 < HL:
                        rdma_l[h + 1, s] = mk("l", h + 1, s)
                        rdma_l[h + 1, s].start()
            store_chunk(prm[2 + h], comm_r[h])
            if h < HL:
                store_chunk(prm[2 + HR + h], comm_l[h])
            for s in range(n_sub):
                rdma_r[h, s].wait_send()
                if h < HL:
                    rdma_l[h, s].wait_send()

    return pl.pallas_call(
        body,
        out_shape=jax.ShapeDtypeStruct((N_DEV * m_per, n), jnp.float32),
        in_specs=[
            pl.BlockSpec(memory_space=pltpu.VMEM),
            pl.BlockSpec(memory_space=pltpu.VMEM),
            pl.BlockSpec(memory_space=pltpu.SMEM),
            pl.BlockSpec(memory_space=pltpu.SMEM),
        ],
        out_specs=pl.BlockSpec(memory_space=pltpu.VMEM),
        scratch_shapes=[
            pltpu.VMEM((HR, m_per, k), jnp.float8_e4m3fn),
            pltpu.VMEM((HL, m_per, k), jnp.float8_e4m3fn),
            pltpu.SemaphoreType.DMA((4,)),
            pltpu.SemaphoreType.DMA((4,)),
            pltpu.SemaphoreType.DMA((4,)),
            pltpu.SemaphoreType.DMA((4,)),
        ],
        compiler_params=pltpu.CompilerParams(collective_id=0),
    )(x8, w16, s, params)


# device time: 103089 ns/iter; 2.4698x vs baseline; 1.0205x over previous
import jax
import jax.numpy as jnp
from jax import lax
from jax.experimental import pallas as pl
from jax.experimental.pallas import tpu as pltpu

N_DEV = 32
H = 16
N_SUB = 4


def _ring_order():
    ring = []
    for p in range(N_DEV):
        if p < 16:
            x, pp = 0, p
        else:
            x, pp = 1, 31 - p
        z = pp // 4
        yy = pp % 4
        y = yy if z % 2 == 0 else 3 - yy
        ring.append(z * 8 + y * 2 + (x if y % 2 == 0 else 1 - x))
    return ring


_RING = _ring_order()
_INV = [0] * N_DEV
for _p, _l in enumerate(_RING):
    _INV[_l] = _p


def _subs_of(direction, h):
    if h < H - 1:
        return range(N_SUB)
    return range(N_SUB // 2) if direction == "r" else range(N_SUB // 2, N_SUB)


def kernel(x, w_mat, scale_x, scale_w):
    m_per, k = x.shape
    n = w_mat.shape[1]

    x8 = x.astype(jnp.float8_e4m3fn)
    w16 = w_mat.astype(jnp.bfloat16)
    s = (scale_x * scale_w).astype(jnp.float32)

    ring_arr = jnp.array(_RING, jnp.int32)
    inv_arr = jnp.array(_INV, jnp.int32)
    my = lax.axis_index("i")
    p = inv_arr[my]
    right = ring_arr[(p + 1) % N_DEV]
    left = ring_arr[(p - 1) % N_DEV]
    origins_r = ring_arr[(p - 1 - jnp.arange(H)) % N_DEV]
    origins_l = ring_arr[(p + 1 + jnp.arange(H - 1)) % N_DEV]
    params = jnp.concatenate(
        [right[None], left[None], origins_r, origins_l]
    ).astype(jnp.int32)

    def body(x_ref, w_ref, s_ref, prm, out_ref,
             comm_r, comm_l, send_r, recv_r, send_l, recv_l):
        rt = prm[0]
        lt = prm[1]

        barrier = pltpu.get_barrier_semaphore()
        for nbr in (lt, rt):
            pl.semaphore_signal(
                barrier, inc=1,
                device_id=(nbr,), device_id_type=pl.DeviceIdType.MESH,
            )
        pl.semaphore_wait(barrier, 2)

        w = w_ref[...]
        scale = s_ref[0]
        sub_m = m_per // N_SUB
        subs = tuple(pl.ds(i * sub_m, sub_m) for i in range(N_SUB))

        def store_rows(origin, row_off, chunk8):
            acc = jnp.dot(
                chunk8.astype(jnp.bfloat16), w,
                preferred_element_type=jnp.float32,
            )
            out_ref[pl.ds(origin * m_per + row_off, chunk8.shape[0]), :] = (
                jnp.maximum(acc * scale, 0.0)
            )

        def mk(direction, h, sb):
            comm, send, recv, tgt = (
                (comm_r, send_r, recv_r, rt) if direction == "r"
                else (comm_l, send_l, recv_l, lt)
            )
            src = x_ref.at[subs[sb]] if h == 0 else comm.at[h - 1, subs[sb]]
            return pltpu.make_async_remote_copy(
                src_ref=src, dst_ref=comm.at[h, subs[sb]],
                send_sem=send.at[sb], recv_sem=recv.at[sb],
                device_id=(tgt,), device_id_type=pl.DeviceIdType.MESH,
            )

        rdma_r = {(0, sb): mk("r", 0, sb) for sb in _subs_of("r", 0)}
        rdma_l = {(0, sb): mk("l", 0, sb) for sb in _subs_of("l", 0)}
        for d in (*rdma_r.values(), *rdma_l.values()):
            d.start()

        store_rows(lax.axis_index("i"), 0, x_ref[...])

        half_rows = (N_SUB // 2) * sub_m
        for h in range(H):
            for sb in range(N_SUB):
                if sb in _subs_of("r", h):
                    rdma_r[h, sb].wait_recv()
                    if h + 1 < H and sb in _subs_of("r", h + 1):
                        rdma_r[h + 1, sb] = mk("r", h + 1, sb)
                        rdma_r[h + 1, sb].start()
                if sb in _subs_of("l", h):
                    rdma_l[h, sb].wait_recv()
                    if h + 1 < H and sb in _subs_of("l", h + 1):
                        rdma_l[h + 1, sb] = mk("l", h + 1, sb)
                        rdma_l[h + 1, sb].start()
            if h < H - 1:
                store_rows(prm[2 + h], 0, comm_r[h])
                store_rows(prm[2 + H + h], 0, comm_l[h])
            else:
                store_rows(prm[2 + h], 0, comm_r[h, :half_rows])
                store_rows(prm[2 + h], half_rows, comm_l[h, half_rows:])
            for sb in _subs_of("r", h):
                rdma_r[h, sb].wait_send()
            for sb in _subs_of("l", h):
                rdma_l[h, sb].wait_send()

    return pl.pallas_call(
        body,
        out_shape=jax.ShapeDtypeStruct((N_DEV * m_per, n), jnp.float32),
        in_specs=[
            pl.BlockSpec(memory_space=pltpu.VMEM),
            pl.BlockSpec(memory_space=pltpu.VMEM),
            pl.BlockSpec(memory_space=pltpu.SMEM),
            pl.BlockSpec(memory_space=pltpu.SMEM),
        ],
        out_specs=pl.BlockSpec(memory_space=pltpu.VMEM),
        scratch_shapes=[
            pltpu.VMEM((H, m_per, k), jnp.float8_e4m3fn),
            pltpu.VMEM((H, m_per, k), jnp.float8_e4m3fn),
            pltpu.SemaphoreType.DMA((N_SUB,)),
            pltpu.SemaphoreType.DMA((N_SUB,)),
            pltpu.SemaphoreType.DMA((N_SUB,)),
            pltpu.SemaphoreType.DMA((N_SUB,)),
        ],
        compiler_params=pltpu.CompilerParams(collective_id=0),
    )(x8, w16, s, params)
